# Initial kernel scaffold; baseline (speedup 1.0000x reference)
import jax
import jax.numpy as jnp
from jax.experimental import pallas as pl


def kernel(projData, B_rows, B_cols, B_vals, cosWeight, fltRamp):
    raise NotImplementedError("write your pallas kernel here")


# speedup vs baseline: 27.4523x; 27.4523x over previous
import functools

import jax
import jax.numpy as jnp
from jax import lax
from jax.experimental import pallas as pl
from jax.experimental.pallas import tpu as pltpu
from jax.experimental.pallas import tpu_sc as plsc

IM = 256
NPIX = IM * IM
NDET = 128
NVIEW = 256
NCOLS = NDET * NVIEW
NNZ = 2097152
BATCH = 4

NCHUNKS_W = 8
NNZ_W = NNZ // NCHUNKS_W
CH = 2048
NSTEPS = NNZ_W // CH
GU = 4


def _filter_body(flt_ref, proj_ref, cos_ref, out_ref):
    x = proj_ref[0]
    cw = cos_ref[...]
    xw = x * cw
    acc = flt_ref[0] * xw[0:NDET, :]
    for t in range(1, 15):
        acc = acc + flt_ref[t] * xw[t:t + NDET, :]
    out_ref[0] = acc


def _tc_filter(proj_pad, cos_pad, flt):
    return pl.pallas_call(
        _filter_body,
        grid=(BATCH,),
        in_specs=[
            pl.BlockSpec(memory_space=pltpu.SMEM),
            pl.BlockSpec((1, 144, NVIEW), lambda b: (b, 0, 0)),
            pl.BlockSpec((144, 1), lambda b: (0, 0)),
        ],
        out_specs=pl.BlockSpec((1, NDET, NVIEW), lambda b: (b, 0, 0)),
        out_shape=jax.ShapeDtypeStruct((BATCH, NDET, NVIEW), jnp.float32),
    )(flt, proj_pad, cos_pad)


def _spmm_body(proj2d, rows, cols, vals, out, table, acc, rbuf, cbuf, vbuf):
    c = lax.axis_index("c")
    s = lax.axis_index("s")
    wid = c * 16 + s
    j = wid // NCHUNKS_W
    i = wid % NCHUNKS_W
    base0 = i * NNZ_W

    pltpu.sync_copy(proj2d.at[j], table)

    zero = jnp.zeros((16,), jnp.float32)

    def zbody(k, _):
        for u in range(8):
            acc[pl.ds(k * 128 + u * 16, 16)] = zero
        return 0

    lax.fori_loop(0, NPIX // 128, zbody, 0)

    def grp(k, _):
        for u in range(GU):
            o = k * (GU * 16) + u * 16
            cv = cbuf[pl.ds(o, 16)]
            rv = rbuf[pl.ds(o, 16)]
            vv = vbuf[pl.ds(o, 16)]
            t = plsc.load_gather(table, [cv])
            plsc.addupdate_scatter(acc, [rv], t * vv)
        return 0

    def chunk(g, _):
        base = base0 + g * CH
        pltpu.sync_copy(rows.at[pl.ds(base, CH)], rbuf)
        pltpu.sync_copy(cols.at[pl.ds(base, CH)], cbuf)
        pltpu.sync_copy(vals.at[pl.ds(base, CH)], vbuf)
        lax.fori_loop(0, CH // (GU * 16), grp, 0)
        return 0

    lax.fori_loop(0, NSTEPS, chunk, 0)

    pltpu.sync_copy(acc, out.at[j, i])


def _sc_spmm(proj2d, rows, cols, vals):
    run = pl.kernel(
        _spmm_body,
        out_type=jax.ShapeDtypeStruct((BATCH, NCHUNKS_W, NPIX), jnp.float32),
        mesh=plsc.VectorSubcoreMesh(core_axis_name="c", subcore_axis_name="s"),
        compiler_params=pltpu.CompilerParams(needs_layout_passes=False),
        scratch_types=[
            pltpu.VMEM((NCOLS,), jnp.float32),
            pltpu.VMEM((NPIX,), jnp.float32),
            pltpu.VMEM((CH,), jnp.int32),
            pltpu.VMEM((CH,), jnp.int32),
            pltpu.VMEM((CH,), jnp.float32),
        ],
    )
    return run(proj2d, rows, cols, vals)


def _combine_body(p_ref, out_ref):
    out_ref[0, 0] = jnp.sum(p_ref[0], axis=0)


def _tc_combine(partial):
    return pl.pallas_call(
        _combine_body,
        grid=(BATCH, 8),
        in_specs=[pl.BlockSpec((1, NCHUNKS_W, NPIX // 8), lambda b, p: (b, 0, p))],
        out_specs=pl.BlockSpec((1, 1, NPIX // 8), lambda b, p: (b, 0, p)),
        out_shape=jax.ShapeDtypeStruct((BATCH, 1, NPIX), jnp.float32),
    )(partial).reshape(BATCH, NPIX)


def kernel(projData, B_rows, B_cols, B_vals, cosWeight, fltRamp):
    B, C, N, K = projData.shape
    proj_pad = jnp.pad(projData.reshape(B * C, N, K), ((0, 0), (7, 9), (0, 0)))
    cos_pad = jnp.pad(cosWeight, (7, 9)).reshape(144, 1)
    proj2d = _tc_filter(proj_pad, cos_pad, fltRamp).reshape(B * C, N * K)
    partial = _sc_spmm(proj2d, B_rows, B_cols, B_vals)
    x2d_t = _tc_combine(partial)
    return x2d_t.T.reshape(B, C, NPIX // K, IM)


# speedup vs baseline: 44.1168x; 1.6070x over previous
import jax
import jax.numpy as jnp
from jax import lax
from jax.experimental import pallas as pl
from jax.experimental.pallas import tpu as pltpu
from jax.experimental.pallas import tpu_sc as plsc

IM = 256
NPIX = IM * IM
NDET = 128
NVIEW = 256
NCOLS = NDET * NVIEW
NNZ = 2097152
BATCH = 4

NCHUNKS_W = 8
NNZ_W = NNZ // NCHUNKS_W
CH = 4096
NSTEPS = NNZ_W // CH
GU = 8


def _filter_body(flt_ref, proj_ref, cos_ref, out_ref):
    x = proj_ref[0]
    cw = cos_ref[...]
    xw = x * cw
    acc = flt_ref[0] * xw[0:NDET, :]
    for t in range(1, 15):
        acc = acc + flt_ref[t] * xw[t:t + NDET, :]
    out_ref[0] = acc


def _tc_filter(proj_pad, cos_pad, flt):
    return pl.pallas_call(
        _filter_body,
        grid=(BATCH,),
        in_specs=[
            pl.BlockSpec(memory_space=pltpu.SMEM),
            pl.BlockSpec((1, 144, NVIEW), lambda b: (b, 0, 0)),
            pl.BlockSpec((144, 1), lambda b: (0, 0)),
        ],
        out_specs=pl.BlockSpec((1, NDET, NVIEW), lambda b: (b, 0, 0)),
        out_shape=jax.ShapeDtypeStruct((BATCH, NDET, NVIEW), jnp.float32),
    )(flt, proj_pad, cos_pad)


_PK_R = 2048
_PK_B = 256


def _pack_body(r_ref, c_ref, out_ref):
    out_ref[...] = r_ref[...] * NCOLS + c_ref[...]


def _tc_pack(rows2d, cols2d):
    return pl.pallas_call(
        _pack_body,
        grid=(_PK_R // _PK_B,),
        in_specs=[
            pl.BlockSpec((_PK_B, 1024), lambda i: (i, 0)),
            pl.BlockSpec((_PK_B, 1024), lambda i: (i, 0)),
        ],
        out_specs=pl.BlockSpec((_PK_B, 1024), lambda i: (i, 0)),
        out_shape=jax.ShapeDtypeStruct((_PK_R, 1024), jnp.int32),
    )(rows2d, cols2d)


def _spmm_body(proj2d, packed, vals, out, table, acc, p0, p1, v0, v1,
               sem_t, sem0, sem1):
    c = lax.axis_index("c")
    s = lax.axis_index("s")
    wid = c * 16 + s
    j = wid // NCHUNKS_W
    i = wid % NCHUNKS_W
    base0 = i * NNZ_W

    cp_t = pltpu.make_async_copy(proj2d.at[j], table, sem_t)
    cp_t.start()

    pltpu.make_async_copy(packed.at[pl.ds(base0, CH)], p0, sem0).start()
    pltpu.make_async_copy(vals.at[pl.ds(base0, CH)], v0, sem0).start()
    pltpu.make_async_copy(packed.at[pl.ds(base0 + CH, CH)], p1, sem1).start()
    pltpu.make_async_copy(vals.at[pl.ds(base0 + CH, CH)], v1, sem1).start()

    zero = jnp.zeros((16,), jnp.float32)

    def zbody(k, _):
        for u in range(8):
            acc[pl.ds(k * 128 + u * 16, 16)] = zero
        return 0

    lax.fori_loop(0, NPIX // 128, zbody, 0)
    cp_t.wait()

    def make_grp(pbuf, vbuf):
        def grp(k, _):
            for u in range(GU):
                o = k * (GU * 16) + u * 16
                pk = pbuf[pl.ds(o, 16)]
                vv = vbuf[pl.ds(o, 16)]
                cv = lax.bitwise_and(pk, NCOLS - 1)
                rv = lax.shift_right_logical(pk, 15)
                t = plsc.load_gather(table, [cv])
                plsc.addupdate_scatter(acc, [rv], t * vv)
            return 0
        return grp

    def pair_body(pair, _):
        for par, pbuf, vbuf, sem in ((0, p0, v0, sem0), (1, p1, v1, sem1)):
            g = pair * 2 + par
            pltpu.make_async_copy(packed.at[pl.ds(base0, CH)], pbuf, sem).wait()
            pltpu.make_async_copy(vals.at[pl.ds(base0, CH)], vbuf, sem).wait()
            lax.fori_loop(0, CH // (GU * 16), make_grp(pbuf, vbuf), 0)

            @pl.when(g + 2 < NSTEPS)
            def _():
                nb = base0 + (g + 2) * CH
                pltpu.make_async_copy(packed.at[pl.ds(nb, CH)], pbuf, sem).start()
                pltpu.make_async_copy(vals.at[pl.ds(nb, CH)], vbuf, sem).start()
        return 0

    lax.fori_loop(0, NSTEPS // 2, pair_body, 0)

    pltpu.sync_copy(acc, out.at[j, i])


def _sc_spmm(proj2d, packed, vals):
    run = pl.kernel(
        _spmm_body,
        out_type=jax.ShapeDtypeStruct((BATCH, NCHUNKS_W, NPIX), jnp.float32),
        mesh=plsc.VectorSubcoreMesh(core_axis_name="c", subcore_axis_name="s"),
        compiler_params=pltpu.CompilerParams(needs_layout_passes=False),
        scratch_types=[
            pltpu.VMEM((NCOLS,), jnp.float32),
            pltpu.VMEM((NPIX,), jnp.float32),
            pltpu.VMEM((CH,), jnp.int32),
            pltpu.VMEM((CH,), jnp.int32),
            pltpu.VMEM((CH,), jnp.float32),
            pltpu.VMEM((CH,), jnp.float32),
            pltpu.SemaphoreType.DMA,
            pltpu.SemaphoreType.DMA,
            pltpu.SemaphoreType.DMA,
        ],
    )
    return run(proj2d, packed, vals)


def _combine_body(p_ref, out_ref):
    out_ref[0, 0] = jnp.sum(p_ref[0], axis=0)


def _tc_combine(partial):
    return pl.pallas_call(
        _combine_body,
        grid=(BATCH, 8),
        in_specs=[pl.BlockSpec((1, NCHUNKS_W, NPIX // 8), lambda b, p: (b, 0, p))],
        out_specs=pl.BlockSpec((1, 1, NPIX // 8), lambda b, p: (b, 0, p)),
        out_shape=jax.ShapeDtypeStruct((BATCH, 1, NPIX), jnp.float32),
    )(partial).reshape(BATCH, NPIX)


def kernel(projData, B_rows, B_cols, B_vals, cosWeight, fltRamp):
    B, C, N, K = projData.shape
    proj_pad = jnp.pad(projData.reshape(B * C, N, K), ((0, 0), (7, 9), (0, 0)))
    cos_pad = jnp.pad(cosWeight, (7, 9)).reshape(144, 1)
    proj2d = _tc_filter(proj_pad, cos_pad, fltRamp).reshape(B * C, N * K)
    packed = _tc_pack(B_rows.reshape(_PK_R, 1024),
                      B_cols.reshape(_PK_R, 1024)).reshape(NNZ)
    partial = _sc_spmm(proj2d, packed, B_vals)
    x2d_t = _tc_combine(partial)
    return x2d_t.T.reshape(B, C, NPIX // K, IM)


# speedup vs baseline: 76.6160x; 1.7367x over previous
import jax
import jax.numpy as jnp
from jax import lax
from jax.experimental import pallas as pl
from jax.experimental.pallas import tpu as pltpu
from jax.experimental.pallas import tpu_sc as plsc

IM = 256
NPIX = IM * IM
NDET = 128
NVIEW = 256
NCOLS = NDET * NVIEW
NNZ = 2097152
BATCH = 4

NCHUNKS_W = 8
NNZ_W = NNZ // NCHUNKS_W
CH = 4096
NSTEPS = NNZ_W // CH
GU = 8


def _filter_body(flt_ref, proj_ref, cos_ref, out_ref):
    x = proj_ref[0]
    cw = cos_ref[...]
    xw = x * cw
    acc = flt_ref[0] * xw[0:NDET, :]
    for t in range(1, 15):
        acc = acc + flt_ref[t] * xw[t:t + NDET, :]
    out_ref[0] = acc


def _tc_filter(proj_pad, cos_pad, flt):
    return pl.pallas_call(
        _filter_body,
        grid=(BATCH,),
        in_specs=[
            pl.BlockSpec(memory_space=pltpu.SMEM),
            pl.BlockSpec((1, 144, NVIEW), lambda b: (b, 0, 0)),
            pl.BlockSpec((144, 1), lambda b: (0, 0)),
        ],
        out_specs=pl.BlockSpec((1, NDET, NVIEW), lambda b: (b, 0, 0)),
        out_shape=jax.ShapeDtypeStruct((BATCH, NDET, NVIEW), jnp.float32),
    )(flt, proj_pad, cos_pad)


_PK_R = 2048
_PK_B = 256


def _pack_body(r_ref, c_ref, out_ref):
    out_ref[...] = r_ref[...] * NCOLS + c_ref[...]


def _tc_pack(rows2d, cols2d):
    return pl.pallas_call(
        _pack_body,
        grid=(_PK_R // _PK_B,),
        in_specs=[
            pl.BlockSpec((_PK_B, 1024), lambda i: (i, 0)),
            pl.BlockSpec((_PK_B, 1024), lambda i: (i, 0)),
        ],
        out_specs=pl.BlockSpec((_PK_B, 1024), lambda i: (i, 0)),
        out_shape=jax.ShapeDtypeStruct((_PK_R, 1024), jnp.int32),
    )(rows2d, cols2d)


def _spmm_body(proj2d, packed, vals, out, table, acc, p0, p1, v0, v1,
               sem_t, sem0, sem1):
    c = lax.axis_index("c")
    s = lax.axis_index("s")
    wid = c * 16 + s
    j = wid // NCHUNKS_W
    i = wid % NCHUNKS_W
    base0 = i * NNZ_W

    cp_t = pltpu.make_async_copy(proj2d.at[j], table, sem_t)
    cp_t.start()

    pltpu.make_async_copy(packed.at[pl.ds(base0, CH)], p0, sem0).start()
    pltpu.make_async_copy(vals.at[pl.ds(base0, CH)], v0, sem0).start()
    pltpu.make_async_copy(packed.at[pl.ds(base0 + CH, CH)], p1, sem1).start()
    pltpu.make_async_copy(vals.at[pl.ds(base0 + CH, CH)], v1, sem1).start()

    zero = jnp.zeros((16,), jnp.float32)

    def zbody(o):
        acc[pl.ds(o, 16)] = zero

    plsc.parallel_loop(0, NPIX, 16, unroll=8)(zbody)
    cp_t.wait()

    def make_grp(pbuf, vbuf):
        def grp(o):
            pk = pbuf[pl.ds(o, 16)]
            vv = vbuf[pl.ds(o, 16)]
            cv = lax.bitwise_and(pk, NCOLS - 1)
            rv = lax.shift_right_logical(pk, 15)
            t = plsc.load_gather(table, [cv])
            plsc.addupdate_scatter(acc, [rv], t * vv)
        return grp

    def pair_body(pair, _):
        for par, pbuf, vbuf, sem in ((0, p0, v0, sem0), (1, p1, v1, sem1)):
            g = pair * 2 + par
            pltpu.make_async_copy(packed.at[pl.ds(base0, CH)], pbuf, sem).wait()
            pltpu.make_async_copy(vals.at[pl.ds(base0, CH)], vbuf, sem).wait()
            plsc.parallel_loop(0, CH, 16, unroll=GU)(make_grp(pbuf, vbuf))

            @pl.when(g + 2 < NSTEPS)
            def _():
                nb = base0 + (g + 2) * CH
                pltpu.make_async_copy(packed.at[pl.ds(nb, CH)], pbuf, sem).start()
                pltpu.make_async_copy(vals.at[pl.ds(nb, CH)], vbuf, sem).start()
        return 0

    lax.fori_loop(0, NSTEPS // 2, pair_body, 0)

    pltpu.sync_copy(acc, out.at[j, i])


def _sc_spmm(proj2d, packed, vals):
    run = pl.kernel(
        _spmm_body,
        out_type=jax.ShapeDtypeStruct((BATCH, NCHUNKS_W, NPIX), jnp.float32),
        mesh=plsc.VectorSubcoreMesh(core_axis_name="c", subcore_axis_name="s"),
        compiler_params=pltpu.CompilerParams(needs_layout_passes=False),
        scratch_types=[
            pltpu.VMEM((NCOLS,), jnp.float32),
            pltpu.VMEM((NPIX,), jnp.float32),
            pltpu.VMEM((CH,), jnp.int32),
            pltpu.VMEM((CH,), jnp.int32),
            pltpu.VMEM((CH,), jnp.float32),
            pltpu.VMEM((CH,), jnp.float32),
            pltpu.SemaphoreType.DMA,
            pltpu.SemaphoreType.DMA,
            pltpu.SemaphoreType.DMA,
        ],
    )
    return run(proj2d, packed, vals)


def _combine_body(p_ref, out_ref):
    out_ref[0, 0] = jnp.sum(p_ref[0], axis=0)


def _tc_combine(partial):
    return pl.pallas_call(
        _combine_body,
        grid=(BATCH, 8),
        in_specs=[pl.BlockSpec((1, NCHUNKS_W, NPIX // 8), lambda b, p: (b, 0, p))],
        out_specs=pl.BlockSpec((1, 1, NPIX // 8), lambda b, p: (b, 0, p)),
        out_shape=jax.ShapeDtypeStruct((BATCH, 1, NPIX), jnp.float32),
    )(partial).reshape(BATCH, NPIX)


def kernel(projData, B_rows, B_cols, B_vals, cosWeight, fltRamp):
    B, C, N, K = projData.shape
    proj_pad = jnp.pad(projData.reshape(B * C, N, K), ((0, 0), (7, 9), (0, 0)))
    cos_pad = jnp.pad(cosWeight, (7, 9)).reshape(144, 1)
    proj2d = _tc_filter(proj_pad, cos_pad, fltRamp).reshape(B * C, N * K)
    packed = _tc_pack(B_rows.reshape(_PK_R, 1024),
                      B_cols.reshape(_PK_R, 1024)).reshape(NNZ)
    partial = _sc_spmm(proj2d, packed, B_vals)
    x2d_t = _tc_combine(partial)
    return x2d_t.T.reshape(B, C, NPIX // K, IM)


# speedup vs baseline: 81.7756x; 1.0673x over previous
import jax
import jax.numpy as jnp
from jax import lax
from jax.experimental import pallas as pl
from jax.experimental.pallas import tpu as pltpu
from jax.experimental.pallas import tpu_sc as plsc

IM = 256
NPIX = IM * IM
NDET = 128
NVIEW = 256
NCOLS = NDET * NVIEW
NNZ = 2097152
BATCH = 4

NCHUNKS_W = 8
NNZ_W = NNZ // NCHUNKS_W
CH = 4096
NSTEPS = NNZ_W // CH
GU = 8

_PK_R = 2048
_PK_B = _PK_R // BATCH


def _filter_pack_body(flt_ref, proj_ref, cos_ref, r_ref, c_ref,
                      out_ref, pk_ref):
    x = proj_ref[0]
    cw = cos_ref[...]
    xw = x * cw
    acc = flt_ref[0] * xw[0:NDET, :]
    for t in range(1, 15):
        acc = acc + flt_ref[t] * xw[t:t + NDET, :]
    out_ref[0] = acc
    pk_ref[...] = r_ref[...] * NCOLS + c_ref[...]


def _tc_filter_pack(proj_pad, cos_pad, flt, rows2d, cols2d):
    return pl.pallas_call(
        _filter_pack_body,
        grid=(BATCH,),
        in_specs=[
            pl.BlockSpec(memory_space=pltpu.SMEM),
            pl.BlockSpec((1, 144, NVIEW), lambda b: (b, 0, 0)),
            pl.BlockSpec((144, 1), lambda b: (0, 0)),
            pl.BlockSpec((_PK_B, 1024), lambda b: (b, 0)),
            pl.BlockSpec((_PK_B, 1024), lambda b: (b, 0)),
        ],
        out_specs=[
            pl.BlockSpec((1, NDET, NVIEW), lambda b: (b, 0, 0)),
            pl.BlockSpec((_PK_B, 1024), lambda b: (b, 0)),
        ],
        out_shape=[
            jax.ShapeDtypeStruct((BATCH, NDET, NVIEW), jnp.float32),
            jax.ShapeDtypeStruct((_PK_R, 1024), jnp.int32),
        ],
    )(flt, proj_pad, cos_pad, rows2d, cols2d)


def _spmm_body(proj2d, packed, vals, out, table, acc, p0, p1, v0, v1,
               sem_t, sem0, sem1):
    c = lax.axis_index("c")
    s = lax.axis_index("s")
    wid = c * 16 + s
    j = wid // NCHUNKS_W
    i = wid % NCHUNKS_W
    base0 = i * NNZ_W

    cp_t = pltpu.make_async_copy(proj2d.at[j], table, sem_t)
    cp_t.start()

    pltpu.make_async_copy(packed.at[pl.ds(base0, CH)], p0, sem0).start()
    pltpu.make_async_copy(vals.at[pl.ds(base0, CH)], v0, sem0).start()
    pltpu.make_async_copy(packed.at[pl.ds(base0 + CH, CH)], p1, sem1).start()
    pltpu.make_async_copy(vals.at[pl.ds(base0 + CH, CH)], v1, sem1).start()

    zero = jnp.zeros((16,), jnp.float32)

    def zbody(o):
        acc[pl.ds(o, 16)] = zero

    plsc.parallel_loop(0, NPIX, 16, unroll=8)(zbody)
    cp_t.wait()

    def make_grp(pbuf, vbuf):
        def grp(o):
            pk = pbuf[pl.ds(o, 16)]
            vv = vbuf[pl.ds(o, 16)]
            cv = lax.bitwise_and(pk, NCOLS - 1)
            rv = lax.shift_right_logical(pk, 15)
            t = plsc.load_gather(table, [cv])
            plsc.addupdate_scatter(acc, [rv], t * vv)
        return grp

    def pair_body(pair, _):
        for par, pbuf, vbuf, sem in ((0, p0, v0, sem0), (1, p1, v1, sem1)):
            g = pair * 2 + par
            pltpu.make_async_copy(packed.at[pl.ds(base0, CH)], pbuf, sem).wait()
            pltpu.make_async_copy(vals.at[pl.ds(base0, CH)], vbuf, sem).wait()
            plsc.parallel_loop(0, CH, 16, unroll=GU)(make_grp(pbuf, vbuf))

            @pl.when(g + 2 < NSTEPS)
            def _():
                nb = base0 + (g + 2) * CH
                pltpu.make_async_copy(packed.at[pl.ds(nb, CH)], pbuf, sem).start()
                pltpu.make_async_copy(vals.at[pl.ds(nb, CH)], vbuf, sem).start()
        return 0

    lax.fori_loop(0, NSTEPS // 2, pair_body, 0)

    pltpu.sync_copy(acc, out.at[j, i])


def _sc_spmm(proj2d, packed, vals):
    run = pl.kernel(
        _spmm_body,
        out_type=jax.ShapeDtypeStruct((BATCH, NCHUNKS_W, NPIX), jnp.float32),
        mesh=plsc.VectorSubcoreMesh(core_axis_name="c", subcore_axis_name="s"),
        compiler_params=pltpu.CompilerParams(needs_layout_passes=False),
        scratch_types=[
            pltpu.VMEM((NCOLS,), jnp.float32),
            pltpu.VMEM((NPIX,), jnp.float32),
            pltpu.VMEM((CH,), jnp.int32),
            pltpu.VMEM((CH,), jnp.int32),
            pltpu.VMEM((CH,), jnp.float32),
            pltpu.VMEM((CH,), jnp.float32),
            pltpu.SemaphoreType.DMA,
            pltpu.SemaphoreType.DMA,
            pltpu.SemaphoreType.DMA,
        ],
    )
    return run(proj2d, packed, vals)


def _combine_body(p_ref, out_ref):
    s = jnp.sum(p_ref[...], axis=1)
    out_ref[...] = s.T


def _tc_combine_t(partial):
    return pl.pallas_call(
        _combine_body,
        grid=(16,),
        in_specs=[pl.BlockSpec((BATCH, NCHUNKS_W, NPIX // 16),
                               lambda p: (0, 0, p))],
        out_specs=pl.BlockSpec((NPIX // 16, BATCH), lambda p: (p, 0)),
        out_shape=jax.ShapeDtypeStruct((NPIX, BATCH), jnp.float32),
    )(partial)


def kernel(projData, B_rows, B_cols, B_vals, cosWeight, fltRamp):
    B, C, N, K = projData.shape
    proj_pad = jnp.pad(projData.reshape(B * C, N, K), ((0, 0), (7, 9), (0, 0)))
    cos_pad = jnp.pad(cosWeight, (7, 9)).reshape(144, 1)
    proj2d, packed = _tc_filter_pack(proj_pad, cos_pad, fltRamp,
                                     B_rows.reshape(_PK_R, 1024),
                                     B_cols.reshape(_PK_R, 1024))
    partial = _sc_spmm(proj2d.reshape(B * C, N * K), packed.reshape(NNZ),
                       B_vals)
    x2d = _tc_combine_t(partial)
    return x2d.reshape(B, C, NPIX // K, IM)


# speedup vs baseline: 94.3234x; 1.1534x over previous
import jax
import jax.numpy as jnp
from jax import lax
from jax.experimental import pallas as pl
from jax.experimental.pallas import tpu as pltpu
from jax.experimental.pallas import tpu_sc as plsc

IM = 256
NPIX = IM * IM
NDET = 128
NVIEW = 256
NCOLS = NDET * NVIEW
NNZ = 2097152
BATCH = 4

NCHUNKS_W = 8
NNZ_W = NNZ // NCHUNKS_W
CH = 4096
NSTEPS = NNZ_W // CH
GU = 8

_PK_R = 2048
_PK_B = _PK_R // BATCH


def _filter_pack_body(flt_ref, proj_ref, cos_ref, r_ref, c_ref,
                      out_ref, pk_ref):
    x = proj_ref[0]
    cw = cos_ref[...]
    xw = x * cw
    acc = flt_ref[0] * xw[0:NDET, :]
    for t in range(1, 15):
        acc = acc + flt_ref[t] * xw[t:t + NDET, :]
    out_ref[0] = acc
    pk_ref[...] = r_ref[...] * NCOLS + c_ref[...]


def _tc_filter_pack(proj_pad, cos_pad, flt, rows2d, cols2d):
    return pl.pallas_call(
        _filter_pack_body,
        grid=(BATCH,),
        in_specs=[
            pl.BlockSpec(memory_space=pltpu.SMEM),
            pl.BlockSpec((1, 144, NVIEW), lambda b: (b, 0, 0)),
            pl.BlockSpec((144, 1), lambda b: (0, 0)),
            pl.BlockSpec((_PK_B, 1024), lambda b: (b, 0)),
            pl.BlockSpec((_PK_B, 1024), lambda b: (b, 0)),
        ],
        out_specs=[
            pl.BlockSpec((1, NDET, NVIEW), lambda b: (b, 0, 0)),
            pl.BlockSpec((_PK_B, 1024), lambda b: (b, 0)),
        ],
        out_shape=[
            jax.ShapeDtypeStruct((BATCH, NDET, NVIEW), jnp.float32),
            jax.ShapeDtypeStruct((_PK_R, 1024), jnp.int32),
        ],
    )(flt, proj_pad, cos_pad, rows2d, cols2d)


def _spmm_body(proj2d, packed, vals, out, table, acc, p0, p1, v0, v1,
               sem_t, sem0, sem1):
    c = lax.axis_index("c")
    s = lax.axis_index("s")
    wid = c * 16 + s
    j = wid // NCHUNKS_W
    i = wid % NCHUNKS_W
    base0 = i * NNZ_W

    cp_t = pltpu.make_async_copy(proj2d.at[j], table, sem_t)
    cp_t.start()

    pltpu.make_async_copy(packed.at[pl.ds(base0, CH)], p0, sem0).start()
    pltpu.make_async_copy(vals.at[pl.ds(base0, CH)], v0, sem0).start()
    pltpu.make_async_copy(packed.at[pl.ds(base0 + CH, CH)], p1, sem1).start()
    pltpu.make_async_copy(vals.at[pl.ds(base0 + CH, CH)], v1, sem1).start()

    zero = jnp.zeros((16,), jnp.float32)

    def zbody(o):
        acc[pl.ds(o, 16)] = zero

    plsc.parallel_loop(0, NPIX, 16, unroll=8)(zbody)
    cp_t.wait()

    def make_grp(pbuf, vbuf):
        def grp(o):
            pk = pbuf[pl.ds(o, 16)]
            vv = vbuf[pl.ds(o, 16)]
            cv = lax.bitwise_and(pk, NCOLS - 1)
            rv = lax.shift_right_logical(pk, 15)
            t = plsc.load_gather(table, [cv])
            plsc.addupdate_scatter(acc, [rv], t * vv)
        return grp

    def pair_body(pair, _):
        for par, pbuf, vbuf, sem in ((0, p0, v0, sem0), (1, p1, v1, sem1)):
            g = pair * 2 + par
            pltpu.make_async_copy(packed.at[pl.ds(base0, CH)], pbuf, sem).wait()
            pltpu.make_async_copy(vals.at[pl.ds(base0, CH)], vbuf, sem).wait()
            plsc.parallel_loop(0, CH, 16, unroll=GU)(make_grp(pbuf, vbuf))

            @pl.when(g + 2 < NSTEPS)
            def _():
                nb = base0 + (g + 2) * CH
                pltpu.make_async_copy(packed.at[pl.ds(nb, CH)], pbuf, sem).start()
                pltpu.make_async_copy(vals.at[pl.ds(nb, CH)], vbuf, sem).start()
        return 0

    lax.fori_loop(0, NSTEPS // 2, pair_body, 0)

    pltpu.sync_copy(acc, out.at[j, i])


def _sc_spmm(proj2d, packed, vals):
    run = pl.kernel(
        _spmm_body,
        out_type=jax.ShapeDtypeStruct((BATCH, NCHUNKS_W, NPIX), jnp.float32),
        mesh=plsc.VectorSubcoreMesh(core_axis_name="c", subcore_axis_name="s"),
        compiler_params=pltpu.CompilerParams(needs_layout_passes=False),
        scratch_types=[
            pltpu.VMEM((NCOLS,), jnp.float32),
            pltpu.VMEM((NPIX,), jnp.float32),
            pltpu.VMEM((CH,), jnp.int32),
            pltpu.VMEM((CH,), jnp.int32),
            pltpu.VMEM((CH,), jnp.float32),
            pltpu.VMEM((CH,), jnp.float32),
            pltpu.SemaphoreType.DMA,
            pltpu.SemaphoreType.DMA,
            pltpu.SemaphoreType.DMA,
        ],
    )
    return run(proj2d, packed, vals)


def _combine_body(p_ref, out_ref):
    s = jnp.sum(p_ref[...], axis=1)
    s3 = s.reshape(BATCH, IM, IM // BATCH)
    out_ref[0, 0] = jnp.transpose(s3, (1, 2, 0)).reshape(IM, IM)


def _tc_combine_t(partial):
    return pl.pallas_call(
        _combine_body,
        grid=(BATCH,),
        in_specs=[pl.BlockSpec((BATCH, NCHUNKS_W, NPIX // BATCH),
                               lambda b: (0, 0, b))],
        out_specs=pl.BlockSpec((1, 1, IM, IM), lambda b: (b, 0, 0, 0)),
        out_shape=jax.ShapeDtypeStruct((BATCH, 1, IM, IM), jnp.float32),
    )(partial)


def kernel(projData, B_rows, B_cols, B_vals, cosWeight, fltRamp):
    B, C, N, K = projData.shape
    proj_pad = jnp.pad(projData.reshape(B * C, N, K), ((0, 0), (7, 9), (0, 0)))
    cos_pad = jnp.pad(cosWeight, (7, 9)).reshape(144, 1)
    proj2d, packed = _tc_filter_pack(proj_pad, cos_pad, fltRamp,
                                     B_rows.reshape(_PK_R, 1024),
                                     B_cols.reshape(_PK_R, 1024))
    partial = _sc_spmm(proj2d.reshape(B * C, N * K), packed.reshape(NNZ),
                       B_vals)
    return _tc_combine_t(partial)


# speedup vs baseline: 114.3371x; 1.2122x over previous
import jax
import jax.numpy as jnp
from jax import lax
from jax.experimental import pallas as pl
from jax.experimental.pallas import tpu as pltpu
from jax.experimental.pallas import tpu_sc as plsc

IM = 256
NPIX = IM * IM
NDET = 128
NVIEW = 256
NCOLS = NDET * NVIEW
NNZ = 2097152
BATCH = 4

NCHUNKS_W = 8
NNZ_W = NNZ // NCHUNKS_W
CH = 4096
NSTEPS = NNZ_W // CH
GU = 8

_PK_R = 2048
_PK_B = _PK_R // BATCH


def _filter_pack_body(flt_ref, proj_ref, cos_ref, r_ref, c_ref,
                      out_ref, pk_ref):
    x = proj_ref[0]
    cw = cos_ref[...]
    xw = x * cw
    acc = flt_ref[0] * xw[0:NDET, :]
    for t in range(1, 15):
        acc = acc + flt_ref[t] * xw[t:t + NDET, :]
    out_ref[0] = acc
    pk_ref[...] = r_ref[...] * NCOLS + c_ref[...]


def _tc_filter_pack(proj_pad, cos_pad, flt, rows, cols):
    return pl.pallas_call(
        _filter_pack_body,
        grid=(BATCH,),
        in_specs=[
            pl.BlockSpec(memory_space=pltpu.SMEM),
            pl.BlockSpec((1, 144, NVIEW), lambda b: (b, 0, 0)),
            pl.BlockSpec((144, 1), lambda b: (0, 0)),
            pl.BlockSpec((NNZ // BATCH // 128, 128), lambda b: (b, 0)),
            pl.BlockSpec((NNZ // BATCH // 128, 128), lambda b: (b, 0)),
        ],
        out_specs=[
            pl.BlockSpec((1, NDET, NVIEW), lambda b: (b, 0, 0)),
            pl.BlockSpec((NNZ // BATCH // 128, 128), lambda b: (b, 0)),
        ],
        out_shape=[
            jax.ShapeDtypeStruct((BATCH, NDET, NVIEW), jnp.float32),
            jax.ShapeDtypeStruct((NNZ // 128, 128), jnp.int32),
        ],
    )(flt, proj_pad, cos_pad, rows, cols)


def _spmm_body(proj2d, packed, vals, out, table, acc, p0, p1, v0, v1,
               sem_t, sem0, sem1):
    c = lax.axis_index("c")
    s = lax.axis_index("s")
    wid = c * 16 + s
    j = wid // NCHUNKS_W
    i = wid % NCHUNKS_W
    base0 = i * NNZ_W

    cp_t = pltpu.make_async_copy(proj2d.at[pl.ds(j * NCOLS, NCOLS)], table,
                                 sem_t)
    cp_t.start()

    pltpu.make_async_copy(packed.at[pl.ds(base0, CH)], p0, sem0).start()
    pltpu.make_async_copy(vals.at[pl.ds(base0, CH)], v0, sem0).start()
    pltpu.make_async_copy(packed.at[pl.ds(base0 + CH, CH)], p1, sem1).start()
    pltpu.make_async_copy(vals.at[pl.ds(base0 + CH, CH)], v1, sem1).start()

    zero = jnp.zeros((16,), jnp.float32)

    def zbody(o):
        acc[pl.ds(o, 16)] = zero

    plsc.parallel_loop(0, NPIX, 16, unroll=8)(zbody)
    cp_t.wait()

    def make_grp(pbuf, vbuf):
        def grp(o):
            pk = pbuf[pl.ds(o, 16)]
            vv = vbuf[pl.ds(o, 16)]
            cv = lax.bitwise_and(pk, NCOLS - 1)
            rv = lax.shift_right_logical(pk, 15)
            t = plsc.load_gather(table, [cv])
            plsc.addupdate_scatter(acc, [rv], t * vv)
        return grp

    def pair_body(pair, _):
        for par, pbuf, vbuf, sem in ((0, p0, v0, sem0), (1, p1, v1, sem1)):
            g = pair * 2 + par
            pltpu.make_async_copy(packed.at[pl.ds(base0, CH)], pbuf, sem).wait()
            pltpu.make_async_copy(vals.at[pl.ds(base0, CH)], vbuf, sem).wait()
            plsc.parallel_loop(0, CH, 16, unroll=GU)(make_grp(pbuf, vbuf))

            @pl.when(g + 2 < NSTEPS)
            def _():
                nb = base0 + (g + 2) * CH
                pltpu.make_async_copy(packed.at[pl.ds(nb, CH)], pbuf, sem).start()
                pltpu.make_async_copy(vals.at[pl.ds(nb, CH)], vbuf, sem).start()
        return 0

    lax.fori_loop(0, NSTEPS // 2, pair_body, 0)

    pltpu.sync_copy(acc, out.at[j, i])


def _sc_spmm(proj2d, packed, vals):
    run = pl.kernel(
        _spmm_body,
        out_type=jax.ShapeDtypeStruct((BATCH, NCHUNKS_W, NPIX), jnp.float32),
        mesh=plsc.VectorSubcoreMesh(core_axis_name="c", subcore_axis_name="s"),
        compiler_params=pltpu.CompilerParams(needs_layout_passes=False),
        scratch_types=[
            pltpu.VMEM((NCOLS,), jnp.float32),
            pltpu.VMEM((NPIX,), jnp.float32),
            pltpu.VMEM((CH,), jnp.int32),
            pltpu.VMEM((CH,), jnp.int32),
            pltpu.VMEM((CH,), jnp.float32),
            pltpu.VMEM((CH,), jnp.float32),
            pltpu.SemaphoreType.DMA,
            pltpu.SemaphoreType.DMA,
            pltpu.SemaphoreType.DMA,
        ],
    )
    return run(proj2d, packed, vals)


def _combine_body(p_ref, out_ref):
    s = jnp.sum(p_ref[...], axis=1)
    s3 = s.reshape(BATCH, IM, IM // BATCH)
    out_ref[0, 0] = jnp.transpose(s3, (1, 2, 0)).reshape(IM, IM)


def _tc_combine_t(partial):
    return pl.pallas_call(
        _combine_body,
        grid=(BATCH,),
        in_specs=[pl.BlockSpec((BATCH, NCHUNKS_W, NPIX // BATCH),
                               lambda b: (0, 0, b))],
        out_specs=pl.BlockSpec((1, 1, IM, IM), lambda b: (b, 0, 0, 0)),
        out_shape=jax.ShapeDtypeStruct((BATCH, 1, IM, IM), jnp.float32),
    )(partial)


def kernel(projData, B_rows, B_cols, B_vals, cosWeight, fltRamp):
    B, C, N, K = projData.shape
    proj_pad = jnp.pad(projData.reshape(B * C, N, K), ((0, 0), (7, 9), (0, 0)))
    cos_pad = jnp.pad(cosWeight, (7, 9)).reshape(144, 1)
    proj3, packed = _tc_filter_pack(proj_pad, cos_pad, fltRamp,
                                    B_rows.reshape(NNZ // 128, 128),
                                    B_cols.reshape(NNZ // 128, 128))
    partial = _sc_spmm(proj3.reshape(B * C * N * K), packed.reshape(NNZ),
                       B_vals)
    return _tc_combine_t(partial)
